# baseline (speedup 1.0000x reference)
import jax
import jax.numpy as jnp
from jax.experimental import pallas as pl
from jax.experimental.pallas import tpu as pltpu

_ROWS = 8192
_COLS = 1024
_BLOCK_ROWS = 512
_GRID = _ROWS // _BLOCK_ROWS


def _l1_body(pred_ref, gt_ref, out_ref, sacc_ref, cacc_ref):
    i = pl.program_id(0)

    @pl.when(i == 0)
    def _init():
        sacc_ref[...] = jnp.zeros_like(sacc_ref)
        cacc_ref[...] = jnp.zeros_like(cacc_ref)

    p = pred_ref[...]
    g = gt_ref[0]
    m = gt_ref[1] > 0.0
    diff = jnp.where(m, jnp.abs(p - g), 0.0)
    cnt = m.astype(jnp.float32)
    sacc_ref[...] += jnp.sum(diff.reshape(_BLOCK_ROWS // 8, 8, _COLS), axis=0)
    cacc_ref[...] += jnp.sum(cnt.reshape(_BLOCK_ROWS // 8, 8, _COLS), axis=0)

    @pl.when(i == _GRID - 1)
    def _fin():
        out_ref[0, 0] = jnp.sum(sacc_ref[...]) / jnp.sum(cacc_ref[...])


def kernel(pred, gt):
    pred2 = pred.reshape(_ROWS, _COLS)
    gt2 = gt.reshape(2, _ROWS, _COLS)
    out = pl.pallas_call(
        _l1_body,
        grid=(_GRID,),
        in_specs=[
            pl.BlockSpec((_BLOCK_ROWS, _COLS), lambda i: (i, 0)),
            pl.BlockSpec((2, _BLOCK_ROWS, _COLS), lambda i: (0, i, 0)),
        ],
        out_specs=pl.BlockSpec(memory_space=pltpu.SMEM),
        out_shape=jax.ShapeDtypeStruct((1, 1), jnp.float32),
        scratch_shapes=[
            pltpu.VMEM((8, _COLS), jnp.float32),
            pltpu.VMEM((8, _COLS), jnp.float32),
        ],
    )(pred2, gt2)
    return out[0, 0]


# speedup vs baseline: 1.0346x; 1.0346x over previous
import jax
import jax.numpy as jnp
from jax.experimental import pallas as pl
from jax.experimental.pallas import tpu as pltpu

_ROWS = 8192
_COLS = 1024
_R = 512
_STEPS = _ROWS // _R
_NBUF = 4


def _l1_body(pred_hbm, gt_hbm, out_ref,
             pbuf, gbuf, mbuf, sacc_ref, cacc_ref, psem, gsem, msem):
    def issue(step):
        b = step % _NBUF
        r = step * _R
        pltpu.make_async_copy(
            pred_hbm.at[pl.ds(r, _R), :], pbuf.at[b], psem.at[b]).start()
        pltpu.make_async_copy(
            gt_hbm.at[0, pl.ds(r, _R), :], gbuf.at[b], gsem.at[b]).start()
        pltpu.make_async_copy(
            gt_hbm.at[1, pl.ds(r, _R), :], mbuf.at[b], msem.at[b]).start()

    def wait(step):
        b = step % _NBUF
        r = step * _R
        pltpu.make_async_copy(
            pred_hbm.at[pl.ds(r, _R), :], pbuf.at[b], psem.at[b]).wait()
        pltpu.make_async_copy(
            gt_hbm.at[0, pl.ds(r, _R), :], gbuf.at[b], gsem.at[b]).wait()
        pltpu.make_async_copy(
            gt_hbm.at[1, pl.ds(r, _R), :], mbuf.at[b], msem.at[b]).wait()

    sacc_ref[...] = jnp.zeros_like(sacc_ref)
    cacc_ref[...] = jnp.zeros_like(cacc_ref)

    for s in range(_NBUF):
        issue(s)
    for s in range(_STEPS):
        wait(s)
        b = s % _NBUF
        p = pbuf[b]
        g = gbuf[b]
        m = mbuf[b] > 0.0
        diff = jnp.where(m, jnp.abs(p - g), 0.0)
        cnt = m.astype(jnp.float32)
        sacc_ref[...] += jnp.sum(diff.reshape(_R // 8, 8, _COLS), axis=0)
        cacc_ref[...] += jnp.sum(cnt.reshape(_R // 8, 8, _COLS), axis=0)
        if s + _NBUF < _STEPS:
            issue(s + _NBUF)

    out_ref[0, 0] = jnp.sum(sacc_ref[...]) / jnp.sum(cacc_ref[...])


def kernel(pred, gt):
    pred2 = pred.reshape(_ROWS, _COLS)
    gt2 = gt.reshape(2, _ROWS, _COLS)
    out = pl.pallas_call(
        _l1_body,
        in_specs=[
            pl.BlockSpec(memory_space=pl.ANY),
            pl.BlockSpec(memory_space=pl.ANY),
        ],
        out_specs=pl.BlockSpec(memory_space=pltpu.SMEM),
        out_shape=jax.ShapeDtypeStruct((1, 1), jnp.float32),
        scratch_shapes=[
            pltpu.VMEM((_NBUF, _R, _COLS), jnp.float32),
            pltpu.VMEM((_NBUF, _R, _COLS), jnp.float32),
            pltpu.VMEM((_NBUF, _R, _COLS), jnp.float32),
            pltpu.VMEM((8, _COLS), jnp.float32),
            pltpu.VMEM((8, _COLS), jnp.float32),
            pltpu.SemaphoreType.DMA((_NBUF,)),
            pltpu.SemaphoreType.DMA((_NBUF,)),
            pltpu.SemaphoreType.DMA((_NBUF,)),
        ],
    )(pred2, gt2)
    return out[0, 0]


# speedup vs baseline: 4.4767x; 4.3270x over previous
import jax
import jax.numpy as jnp
from jax.experimental import pallas as pl
from jax.experimental.pallas import tpu as pltpu

_LEAD = 512
_B = 32
_GRID = _LEAD // _B


def _l1_body(pred_ref, gtd_ref, msk_ref, out_ref, sacc_ref, cacc_ref):
    i = pl.program_id(0)

    @pl.when(i == 0)
    def _init():
        sacc_ref[...] = jnp.zeros_like(sacc_ref)
        cacc_ref[...] = jnp.zeros_like(cacc_ref)

    p = pred_ref[...]
    g = gtd_ref[...]
    m = msk_ref[...] > 0.0
    diff = jnp.where(m, jnp.abs(p - g), 0.0)
    cnt = m.astype(jnp.float32)
    sacc_ref[...] += jnp.sum(diff.reshape(_B * 16, 8, 128), axis=0)
    cacc_ref[...] += jnp.sum(cnt.reshape(_B * 16, 8, 128), axis=0)

    @pl.when(i == _GRID - 1)
    def _fin():
        out_ref[0, 0] = jnp.sum(sacc_ref[...]) / jnp.sum(cacc_ref[...])


def kernel(pred, gt):
    pred3 = pred.reshape(_LEAD, 128, 128)
    gt3 = gt.reshape(2 * _LEAD, 128, 128)
    out = pl.pallas_call(
        _l1_body,
        grid=(_GRID,),
        in_specs=[
            pl.BlockSpec((_B, 128, 128), lambda i: (i, 0, 0)),
            pl.BlockSpec((_B, 128, 128), lambda i: (i, 0, 0)),
            pl.BlockSpec((_B, 128, 128), lambda i: (i + _GRID, 0, 0)),
        ],
        out_specs=pl.BlockSpec(memory_space=pltpu.SMEM),
        out_shape=jax.ShapeDtypeStruct((1, 1), jnp.float32),
        scratch_shapes=[
            pltpu.VMEM((8, 128), jnp.float32),
            pltpu.VMEM((8, 128), jnp.float32),
        ],
    )(pred3, gt3, gt3)
    return out[0, 0]
